# baseline (speedup 1.0000x reference)
import dataclasses
import functools

import jax
import jax.numpy as jnp
from jax import lax
from jax.experimental import pallas as pl
from jax.experimental.pallas import tpu as pltpu
from jax.experimental.pallas import tpu_sc as plsc

VOCAB = 100000
EMB = 128
BATCH = 16384
NEG = 20

NUM_WORKERS = 32
CHUNK = 128

S = 4
BS = BATCH // S
NC_NEG = BS * NEG // (NUM_WORKERS * CHUNK)

_mesh = plsc.VectorSubcoreMesh(core_axis_name="c", subcore_axis_name="s")

_sc_cp = pltpu.CompilerParams()
if "needs_layout_passes" in pltpu.CompilerParams.__dataclass_fields__:
    _sc_cp = dataclasses.replace(_sc_cp, needs_layout_passes=False)


@functools.partial(
    pl.kernel,
    out_type=(
        jax.ShapeDtypeStruct((BS, EMB), jnp.bfloat16),
        jax.ShapeDtypeStruct((BS, EMB), jnp.bfloat16),
        jax.ShapeDtypeStruct((BS * NEG, EMB), jnp.bfloat16),
    ),
    mesh=_mesh,
    compiler_params=_sc_cp,
    scratch_types=[
        pltpu.VMEM((NC_NEG + 4, CHUNK), jnp.int32),
        pltpu.VMEM((CHUNK, EMB), jnp.float32),
        pltpu.VMEM((CHUNK, EMB), jnp.float32),
        pltpu.VMEM((CHUNK, EMB), jnp.float32),
        pltpu.VMEM((CHUNK, EMB), jnp.float32),
        pltpu.VMEM((CHUNK, EMB), jnp.float32),
        pltpu.VMEM((CHUNK, EMB), jnp.float32),
        pltpu.VMEM((CHUNK, EMB), jnp.bfloat16),
        pltpu.VMEM((CHUNK, EMB), jnp.bfloat16),
        pltpu.SemaphoreType.DMA,
        pltpu.SemaphoreType.DMA,
        pltpu.SemaphoreType.DMA,
        pltpu.SemaphoreType.DMA,
        pltpu.SemaphoreType.DMA,
        pltpu.SemaphoreType.DMA,
        pltpu.SemaphoreType.DMA,
        pltpu.SemaphoreType.DMA,
    ],
)
def _sc_gather(center_hbm, context_hbm, idx_hbm,
               v_out, upos_out, uneg_out,
               idx_v, rbc, rbp, rb0, rb1, rb2, rb3, wb0, wb1,
               sgc, sgp, sg0, sg1, sg2, sg3, sw0, sw1):
    wid = lax.axis_index("s") * 2 + lax.axis_index("c")
    rb = (rb0, rb1, rb2, rb3)
    sg = (sg0, sg1, sg2, sg3)
    wb = (wb0, wb1)
    sw = (sw0, sw1)
    n = NC_NEG

    pltpu.sync_copy(idx_hbm.at[pl.ds(wid * (NC_NEG + 4), NC_NEG + 4)], idx_v)

    pltpu.async_copy(center_hbm.at[idx_v.at[0]], rbc, sgc)
    pltpu.async_copy(context_hbm.at[idx_v.at[1]], rbp, sgp)

    base = wid * CHUNK

    def convert(src, dst):
        @pl.loop(0, CHUNK)
        def _(r):
            for c in range(4):
                a = src[r, pl.ds(c * 32, 16)]
                b = src[r, pl.ds(c * 32 + 16, 16)]
                dst[r, pl.ds(c * 32, 32)] = plsc.pack(
                    a, b, format=plsc.PackFormat.INTERLEAVED)

    def gst(j, b):
        pltpu.async_copy(context_hbm.at[idx_v.at[2 + j]], rb[b], sg[b])

    def gwait(j, b):
        pltpu.make_async_copy(context_hbm.at[idx_v.at[2 + j]], rb[b], sg[b]).wait()

    def wst(j, w):
        pltpu.async_copy(
            wb[w], uneg_out.at[pl.ds(base + j * BS, CHUNK)], sw[w])

    def wwait(j, w):
        pltpu.make_async_copy(
            wb[w], uneg_out.at[pl.ds(base + j * BS, CHUNK)], sw[w]).wait()

    gst(0, 0)
    gst(1, 1)
    gst(2, 2)
    for j in (0, 1):
        gwait(j, j)
        convert(rb[j], wb[j])
        gst(j + 3, (j + 3) % 4)
        wst(j, j)

    @pl.loop(0, (n - 8) // 4)
    def _(t):
        for r in range(2, 6):
            j = r + 4 * t
            b = r % 4
            gwait(j, b)
            wwait(j - 2, r % 2)
            convert(rb[b], wb[r % 2])
            gst(j + 3, (r + 3) % 4)
            wst(j, r % 2)

    for j in range(n - 6, n):
        gwait(j, j % 4)
        wwait(j - 2, j % 2)
        convert(rb[j % 4], wb[j % 2])
        if j + 3 < n:
            gst(j + 3, (j + 3) % 4)
        wst(j, j % 2)
    wwait(n - 2, (n - 2) % 2)
    wwait(n - 1, (n - 1) % 2)

    def cpwst(rbx, sgx, out, w):
        pltpu.make_async_copy(
            center_hbm.at[idx_v.at[0]] if rbx is rbc
            else context_hbm.at[idx_v.at[1]], rbx, sgx).wait()
        convert(rbx, wb[w])
        pltpu.async_copy(wb[w], out.at[pl.ds(wid * CHUNK, CHUNK)], sw[w])
        pltpu.make_async_copy(wb[w], out.at[pl.ds(wid * CHUNK, CHUNK)], sw[w]).wait()

    cpwst(rbc, sgc, v_out, 0)
    cpwst(rbp, sgp, upos_out, 1)


def _log_sigmoid(x):
    return jnp.minimum(x, 0.0) - jnp.log(1.0 + jnp.exp(-jnp.abs(x)))


BB = 1024


def _loss_body(v_ref, upos_ref, uneg_ref, e_ref, out_ref):
    i = pl.program_id(0)
    v = v_ref[...]
    parts = [v * upos_ref[...]]
    for k in range(NEG):
        parts.append(uneg_ref[k] * v)
    z = jnp.concatenate(parts, axis=1)
    scores = lax.dot_general(z, e_ref[...], (((1,), (0,)), ((), ())),
                             preferred_element_type=jnp.float32)
    block_sum = jnp.sum(_log_sigmoid(scores)).reshape(1, 1)

    @pl.when(i == 0)
    def _():
        out_ref[...] = jnp.zeros((1, 1), jnp.float32)

    out_ref[...] += block_sum


_loss_call = pl.pallas_call(
    _loss_body,
    grid=(BS // BB,),
    in_specs=[
        pl.BlockSpec((BB, EMB), lambda i: (i, 0)),
        pl.BlockSpec((BB, EMB), lambda i: (i, 0)),
        pl.BlockSpec((NEG, BB, EMB), lambda i: (0, i, 0)),
        pl.BlockSpec(((NEG + 1) * EMB, NEG + 1), lambda i: (0, 0)),
    ],
    out_specs=pl.BlockSpec((1, 1), lambda i: (0, 0)),
    out_shape=jax.ShapeDtypeStruct((1, 1), jnp.float32),
)


def _make_e():
    sign = jnp.concatenate([jnp.ones((1,)), -jnp.ones((NEG,))]).astype(jnp.float32)
    eye = jnp.repeat(jnp.eye(NEG + 1, dtype=jnp.float32), EMB, axis=0)
    return (eye * sign[None, :]).astype(jnp.bfloat16)


def kernel(center_emb, context_emb, center_words, pos_context_words, neg_context_words):
    cw = center_words.astype(jnp.int32).reshape(S, NUM_WORKERS, 1, CHUNK)
    pw = pos_context_words.astype(jnp.int32).reshape(S, NUM_WORKERS, 1, CHUNK)
    nw = (neg_context_words.astype(jnp.int32).T
          .reshape(NEG, S, NUM_WORKERS, CHUNK).transpose(1, 2, 0, 3))
    pad = jnp.zeros((S, NUM_WORKERS, 2, CHUNK), jnp.int32)
    idx_all = jnp.concatenate([cw, pw, nw, pad], axis=2).reshape(
        S, NUM_WORKERS * (NC_NEG + 4), CHUNK)
    e = _make_e()
    total = jnp.zeros((1, 1), jnp.float32)
    for s in range(S):
        v, u_pos, u_neg = _sc_gather(center_emb, context_emb, idx_all[s])
        total = total + _loss_call(v, u_pos, u_neg.reshape(NEG, BS, EMB), e)
    return jnp.reshape(total * (-1.0 / BATCH), ())


# speedup vs baseline: 1.3462x; 1.3462x over previous
import functools

import jax
import jax.numpy as jnp
from jax import lax
from jax.experimental import pallas as pl
from jax.experimental.pallas import tpu as pltpu
from jax.experimental.pallas import tpu_sc as plsc

VOCAB = 100000
EMB = 128
BATCH = 16384
NEG = 20

NUM_WORKERS = 32
CHUNK = 128

S = 4
BS = BATCH // S
NC_NEG = BS * NEG // (NUM_WORKERS * CHUNK)

_mesh = plsc.VectorSubcoreMesh(core_axis_name="c", subcore_axis_name="s")


@functools.partial(
    pl.kernel,
    out_type=(
        jax.ShapeDtypeStruct((BS, EMB), jnp.float32),
        jax.ShapeDtypeStruct((BS, EMB), jnp.float32),
        jax.ShapeDtypeStruct((BS * NEG, EMB), jnp.float32),
    ),
    mesh=_mesh,
    scratch_types=[
        pltpu.VMEM((NC_NEG + 4, CHUNK), jnp.int32),
        pltpu.VMEM((CHUNK, EMB), jnp.float32),
        pltpu.VMEM((CHUNK, EMB), jnp.float32),
        pltpu.VMEM((CHUNK, EMB), jnp.float32),
        pltpu.VMEM((CHUNK, EMB), jnp.float32),
        pltpu.VMEM((CHUNK, EMB), jnp.float32),
        pltpu.VMEM((CHUNK, EMB), jnp.float32),
        pltpu.SemaphoreType.DMA,
        pltpu.SemaphoreType.DMA,
        pltpu.SemaphoreType.DMA,
        pltpu.SemaphoreType.DMA,
        pltpu.SemaphoreType.DMA,
        pltpu.SemaphoreType.DMA,
        pltpu.SemaphoreType.DMA,
        pltpu.SemaphoreType.DMA,
        pltpu.SemaphoreType.DMA,
        pltpu.SemaphoreType.DMA,
        pltpu.SemaphoreType.DMA,
        pltpu.SemaphoreType.DMA,
    ],
)
def _sc_gather(center_hbm, context_hbm, idx_hbm,
               v_out, upos_out, uneg_out,
               idx_v, rbc, rbp, rb0, rb1, rb2, rb3,
               sgc, sgp, swc, swp, sg0, sg1, sg2, sg3, sw0, sw1, sw2, sw3):
    wid = lax.axis_index("s") * 2 + lax.axis_index("c")
    rb = (rb0, rb1, rb2, rb3)
    sg = (sg0, sg1, sg2, sg3)
    sw = (sw0, sw1, sw2, sw3)
    n = NC_NEG

    pltpu.sync_copy(idx_hbm.at[pl.ds(wid * (NC_NEG + 4), NC_NEG + 4)], idx_v)

    pltpu.async_copy(center_hbm.at[idx_v.at[0]], rbc, sgc)
    pltpu.async_copy(context_hbm.at[idx_v.at[1]], rbp, sgp)

    base = wid * CHUNK

    def gst(j, b):
        pltpu.async_copy(context_hbm.at[idx_v.at[2 + j]], rb[b], sg[b])

    def gwait(j, b):
        pltpu.make_async_copy(context_hbm.at[idx_v.at[2 + j]], rb[b], sg[b]).wait()

    def wst(j, b):
        pltpu.async_copy(
            rb[b], uneg_out.at[pl.ds(base + j * BS, CHUNK)], sw[b])

    def wwait(j, b):
        pltpu.make_async_copy(
            rb[b], uneg_out.at[pl.ds(base + j * BS, CHUNK)], sw[b]).wait()

    gst(0, 0)
    gst(1, 1)
    gst(2, 2)
    gwait(0, 0)
    gst(3, 3)
    wst(0, 0)

    @pl.loop(0, (n - 4) // 4)
    def _(t):
        for r in range(1, 5):
            j = r + 4 * t
            b = r % 4
            gwait(j, b)
            wwait(j - 1, (r - 1) % 4)
            gst(j + 3, (r + 3) % 4)
            wst(j, b)

    for r in range(3, 0, -1):
        j = n - r
        b = j % 4
        gwait(j, b)
        wwait(j - 1, (j - 1) % 4)
        wst(j, b)

    pltpu.make_async_copy(center_hbm.at[idx_v.at[0]], rbc, sgc).wait()
    pltpu.async_copy(rbc, v_out.at[pl.ds(wid * CHUNK, CHUNK)], swc)
    pltpu.make_async_copy(context_hbm.at[idx_v.at[1]], rbp, sgp).wait()
    pltpu.async_copy(rbp, upos_out.at[pl.ds(wid * CHUNK, CHUNK)], swp)
    wwait(n - 1, (n - 1) % 4)
    pltpu.make_async_copy(rbc, v_out.at[pl.ds(wid * CHUNK, CHUNK)], swc).wait()
    pltpu.make_async_copy(rbp, upos_out.at[pl.ds(wid * CHUNK, CHUNK)], swp).wait()


def _log_sigmoid(x):
    return jnp.minimum(x, 0.0) - jnp.log(1.0 + jnp.exp(-jnp.abs(x)))


BB = 2048


def _loss_body(v_ref, upos_ref, uneg_ref, e_ref, out_ref):
    i = pl.program_id(0)
    v = v_ref[...]
    parts = [v * upos_ref[...]]
    for k in range(NEG):
        parts.append(uneg_ref[k] * v)
    z = jnp.concatenate(parts, axis=1).astype(jnp.bfloat16)
    scores = lax.dot_general(z, e_ref[...], (((1,), (0,)), ((), ())),
                             preferred_element_type=jnp.float32)
    block_sum = jnp.sum(_log_sigmoid(scores)).reshape(1, 1)

    @pl.when(i == 0)
    def _():
        out_ref[...] = jnp.zeros((1, 1), jnp.float32)

    out_ref[...] += block_sum


_loss_call = pl.pallas_call(
    _loss_body,
    grid=(BS // BB,),
    in_specs=[
        pl.BlockSpec((BB, EMB), lambda i: (i, 0)),
        pl.BlockSpec((BB, EMB), lambda i: (i, 0)),
        pl.BlockSpec((NEG, BB, EMB), lambda i: (0, i, 0)),
        pl.BlockSpec(((NEG + 1) * EMB, NEG + 1), lambda i: (0, 0)),
    ],
    out_specs=pl.BlockSpec((1, 1), lambda i: (0, 0)),
    out_shape=jax.ShapeDtypeStruct((1, 1), jnp.float32),
)


def _make_e():
    sign = jnp.concatenate([jnp.ones((1,)), -jnp.ones((NEG,))]).astype(jnp.float32)
    eye = jnp.repeat(jnp.eye(NEG + 1, dtype=jnp.float32), EMB, axis=0)
    return (eye * sign[None, :]).astype(jnp.bfloat16)


def kernel(center_emb, context_emb, center_words, pos_context_words, neg_context_words):
    cw = center_words.astype(jnp.int32).reshape(S, NUM_WORKERS, 1, CHUNK)
    pw = pos_context_words.astype(jnp.int32).reshape(S, NUM_WORKERS, 1, CHUNK)
    nw = (neg_context_words.astype(jnp.int32).T
          .reshape(NEG, S, NUM_WORKERS, CHUNK).transpose(1, 2, 0, 3))
    pad = jnp.zeros((S, NUM_WORKERS, 2, CHUNK), jnp.int32)
    idx_all = jnp.concatenate([cw, pw, nw, pad], axis=2).reshape(
        S, NUM_WORKERS * (NC_NEG + 4), CHUNK)
    e = _make_e()
    total = jnp.zeros((1, 1), jnp.float32)
    for s in range(S):
        v, u_pos, u_neg = _sc_gather(center_emb, context_emb, idx_all[s])
        total = total + _loss_call(v, u_pos, u_neg.reshape(NEG, BS, EMB), e)
    return jnp.reshape(total * (-1.0 / BATCH), ())


# speedup vs baseline: 1.3532x; 1.0053x over previous
import functools

import jax
import jax.numpy as jnp
from jax import lax
from jax.experimental import pallas as pl
from jax.experimental.pallas import tpu as pltpu
from jax.experimental.pallas import tpu_sc as plsc

VOCAB = 100000
EMB = 128
BATCH = 16384
NEG = 20

NUM_WORKERS = 32
CHUNK = 128

S = 4
BS = BATCH // S
NC_NEG = BS * NEG // (NUM_WORKERS * CHUNK)

_mesh = plsc.VectorSubcoreMesh(core_axis_name="c", subcore_axis_name="s")


@functools.partial(
    pl.kernel,
    out_type=(
        jax.ShapeDtypeStruct((BS, EMB), jnp.float32),
        jax.ShapeDtypeStruct((BS, EMB), jnp.float32),
        jax.ShapeDtypeStruct((BS * NEG, EMB), jnp.float32),
    ),
    mesh=_mesh,
    scratch_types=[
        pltpu.VMEM((NC_NEG + 4, CHUNK), jnp.int32),
        pltpu.VMEM((CHUNK, EMB), jnp.float32),
        pltpu.VMEM((CHUNK, EMB), jnp.float32),
        pltpu.VMEM((CHUNK, EMB), jnp.float32),
        pltpu.VMEM((CHUNK, EMB), jnp.float32),
        pltpu.VMEM((CHUNK, EMB), jnp.float32),
        pltpu.VMEM((CHUNK, EMB), jnp.float32),
        pltpu.SemaphoreType.DMA,
        pltpu.SemaphoreType.DMA,
        pltpu.SemaphoreType.DMA,
        pltpu.SemaphoreType.DMA,
        pltpu.SemaphoreType.DMA,
        pltpu.SemaphoreType.DMA,
        pltpu.SemaphoreType.DMA,
        pltpu.SemaphoreType.DMA,
        pltpu.SemaphoreType.DMA,
        pltpu.SemaphoreType.DMA,
        pltpu.SemaphoreType.DMA,
        pltpu.SemaphoreType.DMA,
    ],
)
def _sc_gather(center_hbm, context_hbm, idx_hbm,
               v_out, upos_out, uneg_out,
               idx_v, rbc, rbp, rb0, rb1, rb2, rb3,
               sgc, sgp, swc, swp, sg0, sg1, sg2, sg3, sw0, sw1, sw2, sw3):
    wid = lax.axis_index("s") * 2 + lax.axis_index("c")
    rb = (rb0, rb1, rb2, rb3)
    sg = (sg0, sg1, sg2, sg3)
    sw = (sw0, sw1, sw2, sw3)
    n = NC_NEG

    pltpu.sync_copy(idx_hbm.at[pl.ds(wid * (NC_NEG + 4), NC_NEG + 4)], idx_v)

    pltpu.async_copy(center_hbm.at[idx_v.at[0]], rbc, sgc)
    pltpu.async_copy(context_hbm.at[idx_v.at[1]], rbp, sgp)

    base = wid * CHUNK

    def gst(j, b):
        pltpu.async_copy(context_hbm.at[idx_v.at[2 + j]], rb[b], sg[b])

    def gwait(j, b):
        pltpu.make_async_copy(context_hbm.at[idx_v.at[2 + j]], rb[b], sg[b]).wait()

    def wst(j, b):
        pltpu.async_copy(
            rb[b], uneg_out.at[pl.ds(base + j * BS, CHUNK)], sw[b])

    def wwait(j, b):
        pltpu.make_async_copy(
            rb[b], uneg_out.at[pl.ds(base + j * BS, CHUNK)], sw[b]).wait()

    gst(0, 0)
    gst(1, 1)
    gst(2, 2)
    gwait(0, 0)
    gst(3, 3)
    wst(0, 0)

    @pl.loop(0, (n - 4) // 4)
    def _(t):
        for r in range(1, 5):
            j = r + 4 * t
            b = r % 4
            gwait(j, b)
            wwait(j - 1, (r - 1) % 4)
            gst(j + 3, (r + 3) % 4)
            wst(j, b)

    for r in range(3, 0, -1):
        j = n - r
        b = j % 4
        gwait(j, b)
        wwait(j - 1, (j - 1) % 4)
        wst(j, b)

    pltpu.make_async_copy(center_hbm.at[idx_v.at[0]], rbc, sgc).wait()
    pltpu.async_copy(rbc, v_out.at[pl.ds(wid * CHUNK, CHUNK)], swc)
    pltpu.make_async_copy(context_hbm.at[idx_v.at[1]], rbp, sgp).wait()
    pltpu.async_copy(rbp, upos_out.at[pl.ds(wid * CHUNK, CHUNK)], swp)
    wwait(n - 1, (n - 1) % 4)
    pltpu.make_async_copy(rbc, v_out.at[pl.ds(wid * CHUNK, CHUNK)], swc).wait()
    pltpu.make_async_copy(rbp, upos_out.at[pl.ds(wid * CHUNK, CHUNK)], swp).wait()


def _log_sigmoid(x):
    return jnp.minimum(x, 0.0) - jnp.log(1.0 + jnp.exp(-jnp.abs(x)))


BB = 1024


def _loss_body(v_ref, upos_ref, uneg_ref, e_ref, out_ref):
    i = pl.program_id(0)
    v = v_ref[...]
    parts = [v * upos_ref[...]]
    for k in range(NEG):
        parts.append(uneg_ref[k] * v)
    z = jnp.concatenate(parts, axis=1).astype(jnp.bfloat16)
    scores = lax.dot_general(z, e_ref[...], (((1,), (0,)), ((), ())),
                             preferred_element_type=jnp.float32)
    block_sum = jnp.sum(_log_sigmoid(scores)).reshape(1, 1)

    @pl.when(i == 0)
    def _():
        out_ref[...] = jnp.zeros((1, 1), jnp.float32)

    out_ref[...] += block_sum


_loss_call = pl.pallas_call(
    _loss_body,
    grid=(BS // BB,),
    in_specs=[
        pl.BlockSpec((BB, EMB), lambda i: (i, 0)),
        pl.BlockSpec((BB, EMB), lambda i: (i, 0)),
        pl.BlockSpec((NEG, BB, EMB), lambda i: (0, i, 0)),
        pl.BlockSpec(((NEG + 1) * EMB, NEG + 1), lambda i: (0, 0)),
    ],
    out_specs=pl.BlockSpec((1, 1), lambda i: (0, 0)),
    out_shape=jax.ShapeDtypeStruct((1, 1), jnp.float32),
)


def _make_e():
    sign = jnp.concatenate([jnp.ones((1,)), -jnp.ones((NEG,))]).astype(jnp.float32)
    eye = jnp.repeat(jnp.eye(NEG + 1, dtype=jnp.float32), EMB, axis=0)
    return (eye * sign[None, :]).astype(jnp.bfloat16)


def kernel(center_emb, context_emb, center_words, pos_context_words, neg_context_words):
    cw = center_words.astype(jnp.int32).reshape(S, NUM_WORKERS, 1, CHUNK)
    pw = pos_context_words.astype(jnp.int32).reshape(S, NUM_WORKERS, 1, CHUNK)
    nw = (neg_context_words.astype(jnp.int32).T
          .reshape(NEG, S, NUM_WORKERS, CHUNK).transpose(1, 2, 0, 3))
    pad = jnp.zeros((S, NUM_WORKERS, 2, CHUNK), jnp.int32)
    idx_all = jnp.concatenate([cw, pw, nw, pad], axis=2).reshape(
        S, NUM_WORKERS * (NC_NEG + 4), CHUNK)
    e = _make_e()
    total = jnp.zeros((1, 1), jnp.float32)
    for s in range(S):
        v, u_pos, u_neg = _sc_gather(center_emb, context_emb, idx_all[s])
        total = total + _loss_call(v, u_pos, u_neg.reshape(NEG, BS, EMB), e)
    return jnp.reshape(total * (-1.0 / BATCH), ())
